# baseline (speedup 1.0000x reference)
import functools

import jax
import jax.numpy as jnp
from jax import lax
from jax.experimental import pallas as pl
from jax.experimental.pallas import tpu as pltpu

_NEG = -3000.0


def _sample_body(vals_ref, idx_ref, par_ref, out_ref):
    B, K = idx_ref.shape
    vals = vals_ref[...]
    tk = par_ref[:, 0:1].astype(jnp.int32)
    tp = par_ref[:, 1:2]
    temp = par_ref[:, 2:3]

    col = lax.broadcasted_iota(jnp.int32, (B, K), 1)

    l = jnp.where(col >= tk, _NEG, vals) / temp

    e = jnp.exp(l - jnp.max(l, axis=-1, keepdims=True))
    z = jnp.sum(e, axis=-1, keepdims=True)
    probs = e / z

    r = lax.broadcasted_iota(jnp.int32, (K, K), 0)
    c = lax.broadcasted_iota(jnp.int32, (K, K), 1)
    cum = jnp.dot(probs, (r <= c).astype(jnp.float32),
                  preferred_element_type=jnp.float32)

    gmin = jnp.min(probs[:, 0:1], axis=0, keepdims=True)
    cut = jnp.maximum(gmin, tp)
    dropped = jnp.logical_and(cum > cut, col != 0)

    mass = jnp.sum(jnp.where(dropped, 0.0, probs), axis=-1, keepdims=True)
    pick = jnp.sum((0.5 * mass > cum).astype(jnp.int32), axis=-1,
                   keepdims=True)
    out_ref[...] = jnp.sum(jnp.where(col == pick, idx_ref[...], 0),
                           axis=-1, keepdims=True)


@jax.jit
def kernel(token_logits, sampling_params):
    B, V = token_logits.shape
    K = 128

    vals, idx = lax.top_k(token_logits, K)

    out = pl.pallas_call(
        _sample_body,
        out_shape=jax.ShapeDtypeStruct((B, 1), jnp.int32),
        grid=(1,),
        in_specs=[
            pl.BlockSpec((B, K), lambda i: (0, 0)),
            pl.BlockSpec((B, K), lambda i: (0, 0)),
            pl.BlockSpec((B, 3), lambda i: (0, 0)),
        ],
        out_specs=pl.BlockSpec((B, 1), lambda i: (0, 0)),
        compiler_params=pltpu.CompilerParams(
            dimension_semantics=("arbitrary",)),
    )(vals.astype(jnp.float32), idx.astype(jnp.int32),
      sampling_params.astype(jnp.float32))

    return out.reshape(-1)


# speedup vs baseline: 3.8609x; 3.8609x over previous
import functools

import jax
import jax.numpy as jnp
from jax import lax
from jax.experimental import pallas as pl
from jax.experimental.pallas import tpu as pltpu

_NEG = -3000.0
_GSZ = 128
_GPC = 128


def _group_max_body(x_ref, out_ref, *, v_total):
    bb, cw = x_ref.shape
    base = pl.program_id(1) * cw
    col = lax.broadcasted_iota(jnp.int32, (bb, cw), 1) + base
    x = jnp.where(col < v_total, x_ref[...], -jnp.inf)
    out_ref[...] = jnp.max(x.reshape(bb, _GPC, _GSZ), axis=2)


def _sample_body(vals_ref, idx_ref, par_ref, out_ref):
    B, K = idx_ref.shape
    vals = vals_ref[...]
    tk = par_ref[:, 0:1].astype(jnp.int32)
    tp = par_ref[:, 1:2]
    temp = par_ref[:, 2:3]

    col = lax.broadcasted_iota(jnp.int32, (B, K), 1)

    l = jnp.where(col >= tk, _NEG, vals) / temp

    e = jnp.exp(l - jnp.max(l, axis=-1, keepdims=True))
    z = jnp.sum(e, axis=-1, keepdims=True)
    probs = e / z

    r = lax.broadcasted_iota(jnp.int32, (K, K), 0)
    c = lax.broadcasted_iota(jnp.int32, (K, K), 1)
    cum = jnp.dot(probs, (r <= c).astype(jnp.float32),
                  preferred_element_type=jnp.float32)

    gmin = jnp.min(probs[:, 0:1], axis=0, keepdims=True)
    cut = jnp.maximum(gmin, tp)
    dropped = jnp.logical_and(cum > cut, col != 0)

    mass = jnp.sum(jnp.where(dropped, 0.0, probs), axis=-1, keepdims=True)
    pick = jnp.sum((0.5 * mass > cum).astype(jnp.int32), axis=-1,
                   keepdims=True)
    out_ref[...] = jnp.sum(jnp.where(col == pick, idx_ref[...], 0),
                           axis=-1, keepdims=True)


@jax.jit
def kernel(token_logits, sampling_params):
    B, V = token_logits.shape
    K = 128
    logits = token_logits.astype(jnp.float32)

    ng = -(-V // _GSZ)
    nc = -(-ng // _GPC)
    ngp = nc * _GPC
    nb = 2 if B % 2 == 0 else 1
    bb = B // nb
    cw = _GPC * _GSZ

    gmax = pl.pallas_call(
        functools.partial(_group_max_body, v_total=V),
        out_shape=jax.ShapeDtypeStruct((B, ngp), jnp.float32),
        grid=(nb, nc),
        in_specs=[pl.BlockSpec((bb, cw), lambda b, c: (b, c))],
        out_specs=pl.BlockSpec((bb, _GPC), lambda b, c: (b, c)),
        compiler_params=pltpu.CompilerParams(
            dimension_semantics=("parallel", "arbitrary")),
    )(logits)

    _, gids = lax.top_k(gmax, K)
    sorted_gids = jnp.sort(gids, axis=1)

    cand = jnp.take_along_axis(
        logits.reshape(B, ng, _GSZ), sorted_gids[:, :, None], axis=1)
    cand = cand.reshape(B, K * _GSZ)

    vals, pos = lax.top_k(cand, K)
    idx = (jnp.take_along_axis(sorted_gids, pos // _GSZ, axis=1) * _GSZ
           + pos % _GSZ)

    out = pl.pallas_call(
        _sample_body,
        out_shape=jax.ShapeDtypeStruct((B, 1), jnp.int32),
        grid=(1,),
        in_specs=[
            pl.BlockSpec((B, K), lambda i: (0, 0)),
            pl.BlockSpec((B, K), lambda i: (0, 0)),
            pl.BlockSpec((B, 3), lambda i: (0, 0)),
        ],
        out_specs=pl.BlockSpec((B, 1), lambda i: (0, 0)),
        compiler_params=pltpu.CompilerParams(
            dimension_semantics=("arbitrary",)),
    )(vals, idx.astype(jnp.int32), sampling_params.astype(jnp.float32))

    return out.reshape(-1)


# speedup vs baseline: 14.3913x; 3.7274x over previous
import functools

import jax
import jax.numpy as jnp
from jax import lax
from jax.experimental import pallas as pl
from jax.experimental.pallas import tpu as pltpu

_NEG = -3000.0
_GSZ = 128
_GPC = 128


def _group_max_body(x_ref, out_ref, *, v_total):
    bb, cw = x_ref.shape
    base = pl.program_id(1) * cw
    col = lax.broadcasted_iota(jnp.int32, (bb, cw), 1) + base
    x = jnp.where(col < v_total, x_ref[...], -jnp.inf)
    out_ref[...] = jnp.max(x.reshape(bb, _GPC, _GSZ), axis=2)


def _sample_body(vals_ref, idx_ref, par_ref, out_ref):
    B, K = idx_ref.shape
    vals = vals_ref[...]
    tk = par_ref[:, 0:1].astype(jnp.int32)
    tp = par_ref[:, 1:2]
    temp = par_ref[:, 2:3]

    col = lax.broadcasted_iota(jnp.int32, (B, K), 1)

    l = jnp.where(col >= tk, _NEG, vals) / temp

    e = jnp.exp(l - jnp.max(l, axis=-1, keepdims=True))
    z = jnp.sum(e, axis=-1, keepdims=True)
    probs = e / z

    r = lax.broadcasted_iota(jnp.int32, (K, K), 0)
    c = lax.broadcasted_iota(jnp.int32, (K, K), 1)
    cum = jnp.dot(probs, (r <= c).astype(jnp.float32),
                  preferred_element_type=jnp.float32)

    gmin = jnp.min(probs[:, 0:1], axis=0, keepdims=True)
    cut = jnp.maximum(gmin, tp)
    dropped = jnp.logical_and(cum > cut, col != 0)

    mass = jnp.sum(jnp.where(dropped, 0.0, probs), axis=-1, keepdims=True)
    pick = jnp.sum((0.5 * mass > cum).astype(jnp.int32), axis=-1,
                   keepdims=True)
    out_ref[...] = jnp.sum(jnp.where(col == pick, idx_ref[...], 0),
                           axis=-1, keepdims=True)


@jax.jit
def kernel(token_logits, sampling_params):
    B, V = token_logits.shape
    K = 128
    logits = token_logits.astype(jnp.float32)

    ng = -(-V // _GSZ)
    nc = -(-ng // _GPC)
    ngp = nc * _GPC
    nb = 2 if B % 2 == 0 else 1
    bb = B // nb
    cw = _GPC * _GSZ

    gmax = pl.pallas_call(
        functools.partial(_group_max_body, v_total=V),
        out_shape=jax.ShapeDtypeStruct((B, ngp), jnp.float32),
        grid=(nb, nc),
        in_specs=[pl.BlockSpec((bb, cw), lambda b, c: (b, c))],
        out_specs=pl.BlockSpec((bb, _GPC), lambda b, c: (b, c)),
        compiler_params=pltpu.CompilerParams(
            dimension_semantics=("parallel", "arbitrary")),
    )(logits)

    _, gids = lax.top_k(gmax, K)
    sorted_gids = jnp.sort(gids, axis=1)

    cand = jnp.take_along_axis(
        logits.reshape(B, ng, _GSZ), sorted_gids[:, :, None], axis=1)
    cand = cand.reshape(B, K * _GSZ)

    g2 = 8
    cand3 = cand.reshape(B, K * _GSZ // g2, g2)
    m2 = jnp.max(cand3, axis=2)
    _, gids2 = lax.top_k(m2, K)
    sorted_gids2 = jnp.sort(gids2, axis=1)
    cand2 = jnp.take_along_axis(
        cand3, sorted_gids2[:, :, None], axis=1).reshape(B, K * g2)

    vals, pos = lax.top_k(cand2, K)
    ci = jnp.take_along_axis(sorted_gids2, pos // g2, axis=1) * g2 + pos % g2
    idx = (jnp.take_along_axis(sorted_gids, ci // _GSZ, axis=1) * _GSZ
           + ci % _GSZ)

    out = pl.pallas_call(
        _sample_body,
        out_shape=jax.ShapeDtypeStruct((B, 1), jnp.int32),
        grid=(1,),
        in_specs=[
            pl.BlockSpec((B, K), lambda i: (0, 0)),
            pl.BlockSpec((B, K), lambda i: (0, 0)),
            pl.BlockSpec((B, 3), lambda i: (0, 0)),
        ],
        out_specs=pl.BlockSpec((B, 1), lambda i: (0, 0)),
        compiler_params=pltpu.CompilerParams(
            dimension_semantics=("arbitrary",)),
    )(vals, idx.astype(jnp.int32), sampling_params.astype(jnp.float32))

    return out.reshape(-1)
